# baseline (speedup 1.0000x reference)
import jax
import jax.numpy as jnp
from jax.experimental import pallas as pl

_N_RATE = 6
_N_YEAR = 91
_N_GENRE = 25
_N_DIRECTOR = 2186
_EMB = 16
_X2_COLS = 2 + _N_GENRE + _N_DIRECTOR
_TPAD = 128
_BM = 1024


def _tc_body(x8_ref, wc_ref, tab_ref, out_ref):
    xb = x8_ref[...]
    idx = xb[:, 0:2].astype(jnp.int32)
    rate_idx = idx[:, 0:1]
    year_idx = idx[:, 1:2] + _TPAD
    iota = jax.lax.broadcasted_iota(jnp.int32, (xb.shape[0], 2 * _TPAD), 1)
    oh = jnp.logical_or(iota == rate_idx, iota == year_idx).astype(jnp.bfloat16)
    emb = jnp.dot(oh, tab_ref[...], preferred_element_type=jnp.float32)

    xf = xb.astype(jnp.bfloat16)
    gd = jnp.dot(xf, wc_ref[...], preferred_element_type=jnp.float32)
    gd = jax.nn.sigmoid(gd)

    out_ref[...] = jnp.concatenate([emb, gd], axis=1)


def kernel(rate_table, year_table, W_genre, W_director, x2):
    B = x2.shape[0]
    x8 = x2.astype(jnp.int16)
    tab = jnp.zeros((2 * _TPAD, 2 * _EMB), jnp.float32)
    tab = tab.at[:_N_RATE, :_EMB].set(rate_table)
    tab = tab.at[_TPAD:_TPAD + _N_YEAR, _EMB:].set(year_table)
    tab = tab.astype(jnp.bfloat16)
    wc = jnp.zeros((_X2_COLS, 2 * _EMB), jnp.float32)
    wc = wc.at[2:2 + _N_GENRE, :_EMB].set(W_genre.T)
    wc = wc.at[2 + _N_GENRE:, _EMB:].set(W_director.T)
    wc = wc.astype(jnp.bfloat16)

    return pl.pallas_call(
        _tc_body,
        grid=(B // _BM,),
        in_specs=[
            pl.BlockSpec((_BM, _X2_COLS), lambda i: (i, 0)),
            pl.BlockSpec((_X2_COLS, 2 * _EMB), lambda i: (0, 0)),
            pl.BlockSpec((2 * _TPAD, 2 * _EMB), lambda i: (0, 0)),
        ],
        out_specs=pl.BlockSpec((_BM, 4 * _EMB), lambda i: (i, 0)),
        out_shape=jax.ShapeDtypeStruct((B, 4 * _EMB), jnp.float32),
    )(x8, wc, tab)


# speedup vs baseline: 1.0349x; 1.0349x over previous
import jax
import jax.numpy as jnp
from jax.experimental import pallas as pl

_N_RATE = 6
_N_YEAR = 91
_N_GENRE = 25
_N_DIRECTOR = 2186
_EMB = 16
_X2_COLS = 2 + _N_GENRE + _N_DIRECTOR
_TPAD = 128
_BM = 1024


def _tc_body(x8_ref, wc_ref, tab_ref, out_ref):
    xb = x8_ref[...]
    idx = xb[:, 0:2]
    rate_idx = idx[:, 0:1]
    year_idx = idx[:, 1:2] + jnp.bfloat16(_TPAD)
    iota = jax.lax.broadcasted_iota(
        jnp.int32, (xb.shape[0], 2 * _TPAD), 1).astype(jnp.bfloat16)
    oh = jnp.logical_or(iota == rate_idx, iota == year_idx).astype(jnp.bfloat16)
    emb = jnp.dot(oh, tab_ref[...], preferred_element_type=jnp.float32)

    gd = jnp.dot(xb, wc_ref[...], preferred_element_type=jnp.float32)
    gd = jax.nn.sigmoid(gd)

    out_ref[...] = jnp.concatenate([emb, gd], axis=1)


def kernel(rate_table, year_table, W_genre, W_director, x2):
    B = x2.shape[0]
    x8 = x2.astype(jnp.bfloat16)
    tab = jnp.zeros((2 * _TPAD, 2 * _EMB), jnp.float32)
    tab = tab.at[:_N_RATE, :_EMB].set(rate_table)
    tab = tab.at[_TPAD:_TPAD + _N_YEAR, _EMB:].set(year_table)
    tab = tab.astype(jnp.bfloat16)
    wc = jnp.zeros((_X2_COLS, 2 * _EMB), jnp.float32)
    wc = wc.at[2:2 + _N_GENRE, :_EMB].set(W_genre.T)
    wc = wc.at[2 + _N_GENRE:, _EMB:].set(W_director.T)
    wc = wc.astype(jnp.bfloat16)

    return pl.pallas_call(
        _tc_body,
        grid=(B // _BM,),
        in_specs=[
            pl.BlockSpec((_BM, _X2_COLS), lambda i: (i, 0)),
            pl.BlockSpec((_X2_COLS, 2 * _EMB), lambda i: (0, 0)),
            pl.BlockSpec((2 * _TPAD, 2 * _EMB), lambda i: (0, 0)),
        ],
        out_specs=pl.BlockSpec((_BM, 4 * _EMB), lambda i: (i, 0)),
        out_shape=jax.ShapeDtypeStruct((B, 4 * _EMB), jnp.float32),
    )(x8, wc, tab)


# speedup vs baseline: 1.5101x; 1.4593x over previous
import jax
import jax.numpy as jnp
from jax.experimental import pallas as pl

_N_RATE = 6
_N_YEAR = 91
_N_GENRE = 25
_N_DIRECTOR = 2186
_EMB = 16
_X2_COLS = 2 + _N_GENRE + _N_DIRECTOR
_TPAD = 128
_BM = 2048


def _tc_body(x8_ref, wc_ref, tab_ref, out_ref):
    xb = x8_ref[...]
    idx = xb[:, 0:2].astype(jnp.int32)
    rate_idx = idx[:, 0:1]
    year_idx = idx[:, 1:2] + _TPAD
    iota = jax.lax.broadcasted_iota(jnp.int32, (xb.shape[0], 2 * _TPAD), 1)
    oh = jnp.logical_or(iota == rate_idx, iota == year_idx).astype(jnp.bfloat16)
    emb = jnp.dot(oh, tab_ref[...], preferred_element_type=jnp.float32)

    xf = xb.astype(jnp.bfloat16)
    gd = jnp.dot(xf, wc_ref[...], preferred_element_type=jnp.float32)
    gd = jax.nn.sigmoid(gd)

    out_ref[...] = jnp.concatenate([emb, gd], axis=1)


def kernel(rate_table, year_table, W_genre, W_director, x2):
    B = x2.shape[0]
    x8 = x2.astype(jnp.int8)
    tab = jnp.zeros((2 * _TPAD, 2 * _EMB), jnp.float32)
    tab = tab.at[:_N_RATE, :_EMB].set(rate_table)
    tab = tab.at[_TPAD:_TPAD + _N_YEAR, _EMB:].set(year_table)
    tab = tab.astype(jnp.bfloat16)
    wc = jnp.zeros((_X2_COLS, 2 * _EMB), jnp.float32)
    wc = wc.at[2:2 + _N_GENRE, :_EMB].set(W_genre.T)
    wc = wc.at[2 + _N_GENRE:, _EMB:].set(W_director.T)
    wc = wc.astype(jnp.bfloat16)

    return pl.pallas_call(
        _tc_body,
        grid=(B // _BM,),
        in_specs=[
            pl.BlockSpec((_BM, _X2_COLS), lambda i: (i, 0)),
            pl.BlockSpec((_X2_COLS, 2 * _EMB), lambda i: (0, 0)),
            pl.BlockSpec((2 * _TPAD, 2 * _EMB), lambda i: (0, 0)),
        ],
        out_specs=pl.BlockSpec((_BM, 4 * _EMB), lambda i: (i, 0)),
        out_shape=jax.ShapeDtypeStruct((B, 4 * _EMB), jnp.float32),
    )(x8, wc, tab)
